# Initial kernel scaffold; baseline (speedup 1.0000x reference)
import jax
import jax.numpy as jnp
from jax.experimental import pallas as pl


def kernel(sentences, input_mu_table, input_cho_table, transition_mu, transition_cho, output_mu, output_cho):
    raise NotImplementedError("write your pallas kernel here")


# speedup vs baseline: 46.2076x; 46.2076x over previous
import jax
import jax.numpy as jnp
from jax.experimental import pallas as pl
from jax.experimental.pallas import tpu as pltpu
import numpy as np

D = 64
L = 20
B = 16
IC = 2
TC = 4
KC = 4
NL = 20
LOG2PI = float(np.log(2.0 * np.pi))


def _gather_body(tok_ref, mu_ref, cho_ref, out_mu_ref, out_cho_ref):
    del tok_ref
    out_mu_ref[...] = mu_ref[...]
    out_cho_ref[...] = cho_ref[...]


def _gather_rows(tokens, mu_table, cho_table):
    n = tokens.shape[0]
    spec = pltpu.PrefetchScalarGridSpec(
        num_scalar_prefetch=1,
        grid=(n,),
        in_specs=[
            pl.BlockSpec((1, 1, IC * D), lambda i, toks: (toks[i], 0, 0)),
            pl.BlockSpec((1, 1, IC * D), lambda i, toks: (toks[i], 0, 0)),
        ],
        out_specs=[
            pl.BlockSpec((1, 1, IC * D), lambda i, toks: (i, 0, 0)),
            pl.BlockSpec((1, 1, IC * D), lambda i, toks: (i, 0, 0)),
        ],
    )
    out_shape = [
        jax.ShapeDtypeStruct((n, 1, IC * D), jnp.float32),
        jax.ShapeDtypeStruct((n, 1, IC * D), jnp.float32),
    ]
    gmu, gcho = pl.pallas_call(_gather_body, grid_spec=spec, out_shape=out_shape)(
        tokens, mu_table.reshape(-1, 1, IC * D), cho_table.reshape(-1, 1, IC * D))
    return gmu.reshape(n, IC * D), gcho.reshape(n, IC * D)


def _main_body(gmu_ref, gcho_ref, tcho_ref, tmu_ref, omu_ref, ocho_ref,
               out_ref, gj_ref, fs_ref, fm_ref, fv_ref, bs_ref, bm_ref, bv_ref):
    f32 = jnp.float32
    eye = (jax.lax.broadcasted_iota(jnp.int32, (D, D), 0)
           == jax.lax.broadcasted_iota(jnp.int32, (D, D), 1)).astype(f32)
    lane = jax.lax.broadcasted_iota(jnp.int32, (1, 1, 2 * D), 2)
    rowi = jax.lax.broadcasted_iota(jnp.int32, (1, D, 1), 1)

    def gj_inv(V, n):
        aug = jnp.concatenate([V, jnp.broadcast_to(eye[None], (n, D, D))], axis=2)
        gj_ref[0:n, :, :] = aug

        def body(j, ld):
            row = gj_ref[0:n, pl.ds(j, 1), :]
            piv = jnp.sum(jnp.where(lane == j, row, 0.0), axis=2, keepdims=True)
            rown = row / piv
            M = gj_ref[0:n, :, :]
            colj = jnp.sum(jnp.where(lane == j, M, 0.0), axis=2, keepdims=True)
            c = colj - jnp.where(rowi == j, 1.0, 0.0)
            gj_ref[0:n, :, :] = M - c * rown
            return ld + jnp.log(piv[:, :, 0])

        ld = jax.lax.fori_loop(0, D, body, jnp.zeros((n, 1), f32))
        inv = gj_ref[0:n, :, D:2 * D]
        return inv, ld

    def mv(M, x):
        return jnp.sum(M * x[:, None, :], axis=2)

    def bmm(a, b):
        return jax.lax.dot_general(a, b, (((2,), (1,)), ((0,), (0,))),
                                   preferred_element_type=f32)

    def logn(diff, inv, ld):
        maha = jnp.sum(diff * mv(inv, diff), axis=1, keepdims=True)
        return -0.5 * (D * LOG2PI + ld + maha)

    def sym(a):
        return 0.5 * (a + jnp.swapaxes(a, 1, 2))

    def top4(real, m_flat, v_flat, n):
        iota = jax.lax.broadcasted_iota(jnp.int32, (n, 1), 0)
        r = real
        ss, ms, vs = [], [], []
        for _ in range(KC):
            mx = jnp.max(r)
            idx = jnp.min(jnp.where(r == mx, iota, n))
            oh = (iota == idx).astype(f32)
            ss.append(jnp.sum(oh * real).reshape(1, 1))
            ms.append(jnp.sum(oh * m_flat, axis=0, keepdims=True))
            vs.append(jnp.sum(oh[:, :, None] * v_flat, axis=0, keepdims=True))
            r = jnp.where(oh > 0, -jnp.inf, r)
        return (jnp.concatenate(ss, 0), jnp.concatenate(ms, 0),
                jnp.concatenate(vs, 0))

    tcho = tcho_ref[...]
    tv = jax.lax.dot_general(tcho, tcho, (((2,), (2,)), ((0,), (0,))),
                             preferred_element_type=f32)
    S_aa = tv[:, :D, :D]
    S_ab = tv[:, :D, D:]
    S_ba = tv[:, D:, :D]
    S_bb = tv[:, D:, D:]
    tmu = tmu_ref[...]
    mu_a = tmu[:, :D]
    mu_b = tmu[:, D:]

    def run_dir(hs_ref, hm_ref, hv_ref, flip):
        V0 = S_aa + eye[None]
        inv0, ld0 = gj_inv(V0, TC)
        diff0 = -mu_a
        z0 = logn(diff0, inv0, ld0)
        K0 = bmm(S_ba, inv0)
        m0 = mu_b + mv(K0, diff0)
        v0 = sym(S_bb - bmm(K0, S_ab))
        s, m, v = top4(z0, m0, v0, TC)
        hs_ref[pl.ds(0, 1)] = s[None]
        hm_ref[pl.ds(0, 1)] = m[None]
        hv_ref[pl.ds(0, 1)] = v[None]

        def step(i, carry):
            s, m, v = carry
            l = (L - 1 - i) if flip else i
            imu = gmu_ref[:, pl.ds(2 * l, 2), :]
            imu = imu.reshape(IC, D)
            icho = gcho_ref[:, pl.ds(2 * l, 2), :]
            iv = (icho * icho).reshape(IC, D)

            diag_iv = iv[:, :, None] * eye[None]
            vs8 = (v[:, None] + diag_iv[None]).reshape(KC * IC, D, D)
            inv1, ld1 = gj_inv(vs8, KC * IC)
            m_t = jnp.broadcast_to(m[:, None], (KC, IC, D)).reshape(KC * IC, D)
            imu_t = jnp.broadcast_to(imu[None], (KC, IC, D)).reshape(KC * IC, D)
            iv_t = jnp.broadcast_to(iv[None], (KC, IC, D)).reshape(KC * IC, D)
            v_t = jnp.broadcast_to(v[:, None], (KC, IC, D, D)).reshape(KC * IC, D, D)
            z1 = logn(m_t - imu_t, inv1, ld1)
            P = bmm(v_t, inv1)
            nvar = sym(P * iv_t[:, None, :])
            nmu = iv_t * mv(inv1, m_t) + mv(P, imu_t)
            s_t = jnp.broadcast_to(s[:, None], (KC, IC, 1)).reshape(KC * IC, 1)
            comb = s_t + z1

            NT = KC * IC * TC
            V2 = (nvar[:, None] + S_aa[None]).reshape(NT, D, D)
            inv2, ld2 = gj_inv(V2, NT)
            mu_a_t = jnp.broadcast_to(mu_a[None], (KC * IC, TC, D)).reshape(NT, D)
            mu_b_t = jnp.broadcast_to(mu_b[None], (KC * IC, TC, D)).reshape(NT, D)
            nmu_t = jnp.broadcast_to(nmu[:, None], (KC * IC, TC, D)).reshape(NT, D)
            diff2 = nmu_t - mu_a_t
            z2 = logn(diff2, inv2, ld2)
            S_ba_t = jnp.broadcast_to(S_ba[None], (KC * IC, TC, D, D)).reshape(NT, D, D)
            S_ab_t = jnp.broadcast_to(S_ab[None], (KC * IC, TC, D, D)).reshape(NT, D, D)
            S_bb_t = jnp.broadcast_to(S_bb[None], (KC * IC, TC, D, D)).reshape(NT, D, D)
            K = bmm(S_ba_t, inv2)
            mo = mu_b_t + mv(K, diff2)
            vo = sym(S_bb_t - bmm(K, S_ab_t))
            comb_t = jnp.broadcast_to(comb[:, None], (KC * IC, TC, 1)).reshape(NT, 1)
            real = comb_t + z2
            s, m, v = top4(real, mo, vo, NT)
            hs_ref[pl.ds(i + 1, 1)] = s[None]
            hm_ref[pl.ds(i + 1, 1)] = m[None]
            hv_ref[pl.ds(i + 1, 1)] = v[None]
            return s, m, v

        jax.lax.fori_loop(0, L - 1, step, (s, m, v))

    run_dir(fs_ref, fm_ref, fv_ref, False)
    run_dir(bs_ref, bm_ref, bv_ref, True)

    omu = omu_ref[...]
    ocho = ocho_ref[...]
    ov = ocho * ocho
    diag_ov = ov[:, :, None] * eye[None]

    def score_step(i, _):
        j = L - 1 - i
        fs = fs_ref[pl.ds(i, 1)].reshape(KC, 1)
        fm = fm_ref[pl.ds(i, 1)].reshape(KC, D)
        fv = fv_ref[pl.ds(i, 1)].reshape(KC, D, D)
        bs = bs_ref[pl.ds(j, 1)].reshape(KC, 1)
        bm = bm_ref[pl.ds(j, 1)].reshape(KC, D)
        bv = bv_ref[pl.ds(j, 1)].reshape(KC, D, D)

        NC = KC * KC
        vs16 = (fv[:, None] + bv[None]).reshape(NC, D, D)
        inv3, ld3 = gj_inv(vs16, NC)
        fm_t = jnp.broadcast_to(fm[:, None], (KC, KC, D)).reshape(NC, D)
        bm_t = jnp.broadcast_to(bm[None], (KC, KC, D)).reshape(NC, D)
        fv_t = jnp.broadcast_to(fv[:, None], (KC, KC, D, D)).reshape(NC, D, D)
        bv_t = jnp.broadcast_to(bv[None], (KC, KC, D, D)).reshape(NC, D, D)
        z3 = logn(fm_t - bm_t, inv3, ld3)
        nmu = mv(bv_t, mv(inv3, fm_t)) + mv(fv_t, mv(inv3, bm_t))
        nvar = sym(bmm(bmm(fv_t, inv3), bv_t))
        fs_t = jnp.broadcast_to(fs[:, None], (KC, KC, 1)).reshape(NC, 1)
        bs_t = jnp.broadcast_to(bs[None], (KC, KC, 1)).reshape(NC, 1)
        sc = fs_t + bs_t + z3

        NCL = NC * NL
        VL = (nvar[:, None] + diag_ov[None]).reshape(NCL, D, D)
        invL, ldL = gj_inv(VL, NCL)
        nmu_t = jnp.broadcast_to(nmu[:, None], (NC, NL, D)).reshape(NCL, D)
        omu_t = jnp.broadcast_to(omu[None], (NC, NL, D)).reshape(NCL, D)
        zL = logn(nmu_t - omu_t, invL, ldL)
        sc_t = jnp.broadcast_to(sc[:, None], (NC, NL, 1)).reshape(NCL, 1)
        tot = (sc_t + zL).reshape(NC, NL)
        mx = jnp.max(tot, axis=0, keepdims=True)
        lse = mx + jnp.log(jnp.sum(jnp.exp(tot - mx), axis=0, keepdims=True))
        out_ref[:, pl.ds(i, 1), :] = lse[None]
        return 0

    jax.lax.fori_loop(0, L, score_step, 0)


def kernel(sentences, input_mu_table, input_cho_table, transition_mu,
           transition_cho, output_mu, output_cho):
    tokens = sentences.reshape(-1).astype(jnp.int32)
    gmu, gcho = _gather_rows(tokens, input_mu_table, input_cho_table)
    gmu = gmu.reshape(B, L * IC, D)
    gcho = gcho.reshape(B, L * IC, D)

    grid = (B,)
    in_specs = [
        pl.BlockSpec((1, L * IC, D), lambda b: (b, 0, 0)),
        pl.BlockSpec((1, L * IC, D), lambda b: (b, 0, 0)),
        pl.BlockSpec((TC, 2 * D, 2 * D), lambda b: (0, 0, 0)),
        pl.BlockSpec((TC, 2 * D), lambda b: (0, 0)),
        pl.BlockSpec((NL, D), lambda b: (0, 0)),
        pl.BlockSpec((NL, D), lambda b: (0, 0)),
    ]
    out_specs = pl.BlockSpec((1, L, NL), lambda b: (b, 0, 0))
    scratch = [
        pltpu.VMEM((KC * KC * NL, D, 2 * D), jnp.float32),
        pltpu.VMEM((L, KC, 1), jnp.float32),
        pltpu.VMEM((L, KC, D), jnp.float32),
        pltpu.VMEM((L, KC, D, D), jnp.float32),
        pltpu.VMEM((L, KC, 1), jnp.float32),
        pltpu.VMEM((L, KC, D), jnp.float32),
        pltpu.VMEM((L, KC, D, D), jnp.float32),
    ]
    out = pl.pallas_call(
        _main_body,
        grid=grid,
        in_specs=in_specs,
        out_specs=out_specs,
        out_shape=jax.ShapeDtypeStruct((B, L, NL), jnp.float32),
        scratch_shapes=scratch,
        compiler_params=pltpu.CompilerParams(
            dimension_semantics=("parallel",)),
    )(gmu, gcho, transition_cho, transition_mu, output_mu, output_cho)
    return out
